# Initial kernel scaffold; baseline (speedup 1.0000x reference)
import jax
import jax.numpy as jnp
from jax.experimental import pallas as pl


def kernel(simmat, dtoks, qtoks):
    raise NotImplementedError("write your pallas kernel here")


# speedup vs baseline: 21.7052x; 21.7052x over previous
import functools

import jax
import jax.numpy as jnp
from jax import lax
from jax.experimental import pallas as pl
from jax.experimental.pallas import tpu as pltpu
from jax.experimental.pallas import tpu_sc as plsc

_BINS = 30
_B, _C, _Q, _D = 64, 4, 32, 2048
_ROWS = _B * _C * _Q
_HPAD = 32
_NC, _NS, _L = 2, 16, 16
_NW = _NC * _NS
_RPW = _ROWS // _NW
_GROUP = _L
_NGROUPS = _RPW // _GROUP
_LN2 = 0.6931471805599453


def _log_approx(x):
    bits = lax.bitcast_convert_type(x, jnp.int32)
    e = lax.shift_right_logical(bits, 23) - 127
    mbits = (bits & 0x7FFFFF) | 0x3F800000
    m = lax.bitcast_convert_type(mbits, jnp.float32)
    big = m > 1.4142135
    m = jnp.where(big, m * 0.5, m)
    e = jnp.where(big, e + 1, e)
    z = (m - 1.0) / (m + 1.0)
    z2 = z * z
    p = 1.0 + z2 * (1.0 / 3.0 + z2 * (0.2 + z2 * (1.0 / 7.0)))
    return e.astype(jnp.float32) * _LN2 + 2.0 * z * p


def _tec_body(sim_hbm, dt_hbm, qt_hbm, out_hbm, buf, dtb, qtb, hist):
    wid = lax.axis_index("c") * _NS + lax.axis_index("s")
    lane = lax.iota(jnp.int32, _L)
    dbase = lane * _D
    rowbase = lane * _HPAD
    ones = jnp.ones((_L,), jnp.float32)
    zeros = jnp.zeros((_L,), jnp.float32)

    def group_body(g, carry):
        r0 = wid * _RPW + g * _GROUP
        b = r0 // (_C * _Q)
        q0 = lax.rem(r0, _Q)
        pltpu.sync_copy(sim_hbm.at[pl.ds(r0 * _D, _GROUP * _D)], buf)
        pltpu.sync_copy(dt_hbm.at[pl.ds(b * _D, _D)], dtb)
        pltpu.sync_copy(qt_hbm.at[pl.ds(b * _Q + q0, _L)], qtb)
        qm = qtb[...] != -1
        for i in range(_GROUP * _HPAD // _L):
            hist[pl.ds(i * _L, _L)] = zeros

        def d_body(dd, c):
            val = plsc.load_gather(buf, [dbase + dd])
            dt = plsc.load_gather(dtb, [jnp.full((_L,), dd, jnp.int32)])
            m = jnp.logical_and(dt != -1, qm)
            bidx = ((val + 1.000001) * 14.5).astype(jnp.int32)
            bidx = jnp.clip(bidx, 0, _HPAD - 1)
            plsc.addupdate_scatter(hist, [rowbase + bidx], ones, mask=m)
            return c

        lax.fori_loop(0, _D, d_body, 0)
        for i in range(_GROUP * _HPAD // _L):
            h = hist[pl.ds(i * _L, _L)]
            hist[pl.ds(i * _L, _L)] = _log_approx(h + 1e-5)
        pltpu.sync_copy(hist, out_hbm.at[pl.ds(r0 * _HPAD, _GROUP * _HPAD)])
        return carry

    lax.fori_loop(0, _NGROUPS, group_body, 0)


_sc_hist = functools.partial(
    pl.kernel,
    out_type=jax.ShapeDtypeStruct((_ROWS * _HPAD,), jnp.float32),
    mesh=plsc.VectorSubcoreMesh(
        core_axis_name="c", subcore_axis_name="s",
        num_cores=_NC, num_subcores=_NS),
    scratch_types=[
        pltpu.VMEM((_GROUP * _D,), jnp.float32),
        pltpu.VMEM((_D,), jnp.int32),
        pltpu.VMEM((_L,), jnp.int32),
        pltpu.VMEM((_GROUP * _HPAD,), jnp.float32),
    ],
    compiler_params=pltpu.CompilerParams(needs_layout_passes=False),
)(_tec_body)


def kernel(simmat, dtoks, qtoks):
    sim = simmat.reshape(_ROWS * _D)
    dt = dtoks.astype(jnp.int32).reshape(_B * _D)
    qt = qtoks.astype(jnp.int32).reshape(_B * _Q)
    out = _sc_hist(sim, dt, qt)
    out = out.reshape(_ROWS, _HPAD)[:, :_BINS]
    return out.reshape(_B, _C, _Q, _BINS)


# speedup vs baseline: 25.6837x; 1.1833x over previous
import functools

import jax
import jax.numpy as jnp
from jax import lax
from jax.experimental import pallas as pl
from jax.experimental.pallas import tpu as pltpu
from jax.experimental.pallas import tpu_sc as plsc

_BINS = 30
_B, _C, _Q, _D = 64, 4, 32, 2048
_ROWS = _B * _C * _Q
_HPAD = 32
_NC, _NS, _L = 2, 16, 16
_NW = _NC * _NS
_RPW = _ROWS // _NW
_GROUP = _L
_NGROUPS = _RPW // _GROUP
_HWORDS = _GROUP * _HPAD
_MLPAD = _D + _L
_UNROLL = 16
_LN2 = 0.6931471805599453


def _log_approx(x):
    bits = lax.bitcast_convert_type(x, jnp.int32)
    e = lax.shift_right_logical(bits, 23) - 127
    mbits = (bits & 0x7FFFFF) | 0x3F800000
    m = lax.bitcast_convert_type(mbits, jnp.float32)
    big = m > 1.4142135
    m = jnp.where(big, m * 0.5, m)
    e = jnp.where(big, e + 1, e)
    z = (m - 1.0) / (m + 1.0)
    z2 = z * z
    p = 1.0 + z2 * (1.0 / 3.0 + z2 * (0.2 + z2 * (1.0 / 7.0)))
    return e.astype(jnp.float32) * _LN2 + 2.0 * z * p


def _bin_of(val):
    return ((val + 1.000001) * 14.5).astype(jnp.int32)


def _tec_body(sim_hbm, dt_hbm, qt_hbm, out_hbm, buf, dtb, qtb, mlist, hist,
              sem0, sem1):
    wid = lax.axis_index("c") * _NS + lax.axis_index("s")
    lane = lax.iota(jnp.int32, _L)
    dbase = lane * _D
    rowbase = lane * _HPAD
    ones = jnp.ones((_L,), jnp.float32)
    neg_ones = -ones
    zeros = jnp.zeros((_L,), jnp.float32)
    row0 = wid * _RPW
    sems = [sem0, sem1]

    pltpu.sync_copy(dt_hbm.at[pl.ds(wid * 2 * _D, 2 * _D)], dtb)
    pltpu.sync_copy(qt_hbm.at[pl.ds(wid * 2 * _Q, 2 * _Q)], qtb)
    mcnt = []
    for bs in range(2):
        def chunk(k, off, bs=bs):
            dtv = dtb[pl.ds(bs * _D + k * _L, _L)]
            m = dtv == -1
            didx = k * _L + lane
            plsc.store_compressed(mlist.at[pl.ds(bs * _MLPAD + off, _L)],
                                  didx, mask=m)
            return off + jnp.sum(m.astype(jnp.int32))
        mcnt.append(lax.fori_loop(0, _D // _L, chunk, 0))

    def dma_group(g, slot):
        return pltpu.make_async_copy(
            sim_hbm.at[pl.ds((row0 + g * _GROUP) * _D, _GROUP * _D)],
            buf.at[pl.ds(slot * _GROUP * _D, _GROUP * _D)],
            sems[slot])

    def issue(g, slot):
        dma_group(g, slot).start()

    issue(0, 0)

    def one_group(g, slot, carry):
        dma_group(g, slot).wait()

        @pl.when(g + 1 < _NGROUPS)
        def _():
            issue(g + 1, 1 - slot)

        bs = g // 8
        q0 = slot * _L
        sbase = slot * _GROUP * _D
        qm = qtb[pl.ds(bs * _Q + q0, _L)] != -1
        nmask = jnp.where(bs == 0, carry[0], carry[1])

        for i in range(_HWORDS // _L):
            hist[pl.ds(i * _L, _L)] = zeros

        gbase = dbase + sbase

        def d_body(k, c):
            base = gbase + k * _UNROLL
            for j in range(_UNROLL):
                val = plsc.load_gather(buf, [base + j])
                plsc.addupdate_scatter(
                    hist, [rowbase + _bin_of(val)], ones, mask=qm)
            return c

        lax.fori_loop(0, _D // _UNROLL, d_body, 0)

        def corr(j, c):
            dvec = plsc.load_gather(
                mlist, [jnp.full((_L,), bs * _MLPAD, jnp.int32) + j])
            val = plsc.load_gather(buf, [gbase + dvec])
            plsc.addupdate_scatter(
                hist, [rowbase + _bin_of(val)], neg_ones, mask=qm)
            return c

        lax.fori_loop(0, nmask, corr, 0)

        for i in range(_HWORDS // _L):
            h = hist[pl.ds(i * _L, _L)]
            hist[pl.ds(i * _L, _L)] = _log_approx(h + 1e-5)
        pltpu.sync_copy(
            hist, out_hbm.at[pl.ds((row0 + g * _GROUP) * _HPAD, _HWORDS)])

    def group_pair(gg, carry):
        for p in range(2):
            one_group(gg * 2 + p, p, carry)
        return carry

    lax.fori_loop(0, _NGROUPS // 2, group_pair, (mcnt[0], mcnt[1]))


_sc_hist = functools.partial(
    pl.kernel,
    out_type=jax.ShapeDtypeStruct((_ROWS * _HPAD,), jnp.float32),
    mesh=plsc.VectorSubcoreMesh(
        core_axis_name="c", subcore_axis_name="s",
        num_cores=_NC, num_subcores=_NS),
    scratch_types=[
        pltpu.VMEM((2 * _GROUP * _D,), jnp.float32),
        pltpu.VMEM((2 * _D,), jnp.int32),
        pltpu.VMEM((2 * _Q,), jnp.int32),
        pltpu.VMEM((2 * _MLPAD,), jnp.int32),
        pltpu.VMEM((_HWORDS,), jnp.float32),
        pltpu.SemaphoreType.DMA,
        pltpu.SemaphoreType.DMA,
    ],
    compiler_params=pltpu.CompilerParams(needs_layout_passes=False),
)(_tec_body)


def kernel(simmat, dtoks, qtoks):
    sim = simmat.reshape(_ROWS * _D)
    dt = dtoks.astype(jnp.int32).reshape(_B * _D)
    qt = qtoks.astype(jnp.int32).reshape(_B * _Q)
    out = _sc_hist(sim, dt, qt)
    out = out.reshape(_ROWS, _HPAD)[:, :_BINS]
    return out.reshape(_B, _C, _Q, _BINS)
